# baseline (speedup 1.0000x reference)
import functools
import math

import jax
import jax.numpy as jnp
from jax import lax
from jax.experimental import pallas as pl
from jax.experimental.pallas import tpu as pltpu
from jax.experimental.pallas import tpu_sc as plsc

S = 2048
D = 768
E = 8
F = 3072
C = 512
EC = E * C
SRC_PAD = EC + 8

NC = 2
NS = 16
NW = NC * NS

_MESH = plsc.VectorSubcoreMesh(
    core_axis_name="c", subcore_axis_name="s", num_cores=NC, num_subcores=NS
)


def _take16(x, idx):
    return lax.gather(
        x, idx[:, None],
        lax.GatherDimensionNumbers(
            offset_dims=(), collapsed_slice_dims=(0,), start_index_map=(0,)),
        slice_sizes=(1,),
        mode=lax.GatherScatterMode.PROMISE_IN_BOUNDS,
    )


def _gate_route_body(tok_ref, gw_ref, slotsc_ref, slot_ref, wf_ref):
    logits = lax.dot_general(
        tok_ref[...], gw_ref[...],
        (((1,), (1,)), ((), ())),
        preferred_element_type=jnp.float32,
    )
    lmax = jnp.max(logits, axis=1, keepdims=True)
    wsum = jnp.sum(jnp.exp(logits - lmax), axis=1, keepdims=True)
    weight = 1.0 / wsum
    eidx = lax.broadcasted_iota(jnp.int32, (S, E), 1)
    top1 = jnp.min(jnp.where(logits == lmax, eidx, E), axis=1, keepdims=True)
    oh = (eidx == top1).astype(jnp.float32)
    row = lax.broadcasted_iota(jnp.int32, (S, S), 0)
    col = lax.broadcasted_iota(jnp.int32, (S, S), 1)
    tri = (col <= row).astype(jnp.float32)
    cums = jnp.dot(tri, oh, preferred_element_type=jnp.float32)
    rank = jnp.sum(oh * cums, axis=1, keepdims=True).astype(jnp.int32) - 1
    kept = rank < C
    slot = top1 * C + rank
    slotsc_ref[...] = jnp.where(kept, slot, EC)
    slot_ref[...] = jnp.where(kept, slot, 0)
    wf_ref[...] = jnp.where(kept, weight, 0.0)


_gate_route = pl.pallas_call(
    _gate_route_body,
    out_shape=(
        jax.ShapeDtypeStruct((S, 1), jnp.int32),
        jax.ShapeDtypeStruct((S, 1), jnp.int32),
        jax.ShapeDtypeStruct((S, 1), jnp.float32),
    ),
)


_ROWS_S = S // NW


@functools.partial(
    pl.kernel,
    out_type=jax.ShapeDtypeStruct((SRC_PAD,), jnp.int32),
    mesh=_MESH,
    scratch_types=[
        pltpu.VMEM((_ROWS_S,), jnp.int32),
        pltpu.VMEM((_ROWS_S,), jnp.int32),
        pltpu.SemaphoreType.DMA,
    ],
)
def _scatter_src(slotsc_hbm, src_hbm, idx_v, ids_v, sem):
    wid = lax.axis_index("s") * NC + lax.axis_index("c")
    base = wid * _ROWS_S
    pltpu.sync_copy(slotsc_hbm.at[pl.ds(base, _ROWS_S)], idx_v)
    iota = lax.iota(jnp.int32, 16)
    for j in range(_ROWS_S // 16):
        ids_v[pl.ds(j * 16, 16)] = base + j * 16 + iota
    pltpu.async_copy(ids_v, src_hbm.at[idx_v], sem).wait()


_ROWS_D = EC // NW


@functools.partial(
    pl.kernel,
    out_type=jax.ShapeDtypeStruct((EC, D), jnp.float32),
    mesh=_MESH,
    scratch_types=[
        pltpu.VMEM((_ROWS_D,), jnp.int32),
        pltpu.VMEM((_ROWS_D, D), jnp.float32),
        pltpu.SemaphoreType.DMA,
    ],
)
def _dispatch(tok_hbm, src_hbm, disp_hbm, idx_v, rows_v, sem):
    wid = lax.axis_index("s") * NC + lax.axis_index("c")
    base = wid * _ROWS_D
    pltpu.sync_copy(src_hbm.at[pl.ds(base, _ROWS_D)], idx_v)
    for j in range(_ROWS_D // 16):
        t = idx_v[pl.ds(j * 16, 16)]
        idx_v[pl.ds(j * 16, 16)] = jnp.minimum(jnp.maximum(t, 0), S - 1)
    pltpu.async_copy(tok_hbm.at[idx_v], rows_v, sem).wait()
    pltpu.sync_copy(rows_v, disp_hbm.at[pl.ds(base, _ROWS_D)])


_F_BLK = 768


def _gelu(x):
    c = math.sqrt(2.0 / math.pi)
    return 0.5 * x * (1.0 + jnp.tanh(c * (x + 0.044715 * (x * x * x))))


def _ffn_body(disp_ref, w1_ref, w2_ref, out_ref):
    fb = pl.program_id(1)
    x = disp_ref[0]
    h = _gelu(jnp.dot(x, w1_ref[0], preferred_element_type=jnp.float32))
    contrib = jnp.dot(h, w2_ref[0], preferred_element_type=jnp.float32)

    @pl.when(fb == 0)
    def _():
        out_ref[...] = jnp.zeros_like(out_ref)

    out_ref[...] += contrib[None]


_ffn = pl.pallas_call(
    _ffn_body,
    grid=(E, F // _F_BLK),
    in_specs=[
        pl.BlockSpec((1, C, D), lambda e, fb: (e, 0, 0)),
        pl.BlockSpec((1, D, _F_BLK), lambda e, fb: (e, 0, fb)),
        pl.BlockSpec((1, _F_BLK, D), lambda e, fb: (e, fb, 0)),
    ],
    out_specs=pl.BlockSpec((1, C, D), lambda e, fb: (e, 0, 0)),
    out_shape=jax.ShapeDtypeStruct((E, C, D), jnp.float32),
)


_ROWS_C = S // NW


@functools.partial(
    pl.kernel,
    out_type=jax.ShapeDtypeStruct((S, D), jnp.float32),
    mesh=_MESH,
    scratch_types=[
        pltpu.VMEM((_ROWS_C,), jnp.int32),
        pltpu.VMEM((_ROWS_C,), jnp.float32),
        pltpu.VMEM((_ROWS_C, D), jnp.float32),
        pltpu.SemaphoreType.DMA,
    ],
)
def _combine(eo_hbm, slot_hbm, wf_hbm, out_hbm, idx_v, w_v, rows_v, sem):
    wid = lax.axis_index("s") * NC + lax.axis_index("c")
    base = wid * _ROWS_C
    pltpu.sync_copy(slot_hbm.at[pl.ds(base, _ROWS_C)], idx_v)
    pltpu.sync_copy(wf_hbm.at[pl.ds(base, _ROWS_C)], w_v)
    pltpu.async_copy(eo_hbm.at[idx_v], rows_v, sem).wait()

    def chunk(jj, carry):
        w16 = w_v[pl.ds(jj * 16, 16)]
        for r in range(16):
            wb = _take16(w16, jnp.full((16,), r, jnp.int32))
            i = jj * 16 + r

            def col(k, carry2):
                rows_v[i, pl.ds(k * 16, 16)] = rows_v[i, pl.ds(k * 16, 16)] * wb
                return carry2

            lax.fori_loop(0, D // 16, col, 0)
        return carry

    lax.fori_loop(0, _ROWS_C // 16, chunk, 0)
    pltpu.sync_copy(rows_v, out_hbm.at[pl.ds(base, _ROWS_C)])


def kernel(inputs, gate_w, w1, w2):
    tokens = inputs.reshape(S, D)
    slotsc, slot, wf = _gate_route(tokens, gate_w)
    src = _scatter_src(slotsc.reshape(S))
    disp = _dispatch(tokens, src)
    eo = _ffn(disp.reshape(E, C, D), w1, w2).reshape(EC, D)
    out = _combine(eo, slot.reshape(S), wf.reshape(S))
    return out.reshape(inputs.shape)


# speedup vs baseline: 1.0605x; 1.0605x over previous
import functools
import math

import jax
import jax.numpy as jnp
from jax import lax
from jax.experimental import pallas as pl
from jax.experimental.pallas import tpu as pltpu
from jax.experimental.pallas import tpu_sc as plsc

S = 2048
D = 768
E = 8
F = 3072
C = 512
EC = E * C
DISP_PAD = EC + 8

NC = 2
NS = 16
NW = NC * NS

_MESH = plsc.VectorSubcoreMesh(
    core_axis_name="c", subcore_axis_name="s", num_cores=NC, num_subcores=NS
)


def _take16(x, idx):
    return lax.gather(
        x, idx[:, None],
        lax.GatherDimensionNumbers(
            offset_dims=(), collapsed_slice_dims=(0,), start_index_map=(0,)),
        slice_sizes=(1,),
        mode=lax.GatherScatterMode.PROMISE_IN_BOUNDS,
    )


def _gate_route_body(tok_ref, gw_ref, slotd_ref, slot_ref, wf_ref):
    logits = lax.dot_general(
        tok_ref[...], gw_ref[...],
        (((1,), (1,)), ((), ())),
        preferred_element_type=jnp.float32,
    )
    lmax = jnp.max(logits, axis=1, keepdims=True)
    wsum = jnp.sum(jnp.exp(logits - lmax), axis=1, keepdims=True)
    weight = 1.0 / wsum
    eidx = lax.broadcasted_iota(jnp.int32, (S, E), 1)
    top1 = jnp.min(jnp.where(logits == lmax, eidx, E), axis=1, keepdims=True)
    oh = (eidx == top1).astype(jnp.float32)
    row = lax.broadcasted_iota(jnp.int32, (S, S), 0)
    col = lax.broadcasted_iota(jnp.int32, (S, S), 1)
    tri = (col <= row).astype(jnp.float32)
    cums = jnp.dot(tri, oh, preferred_element_type=jnp.float32)
    rank = jnp.sum(oh * cums, axis=1, keepdims=True).astype(jnp.int32) - 1
    kept = rank < C
    slot = top1 * C + rank
    slotd_ref[...] = jnp.where(kept, slot, EC)
    slot_ref[...] = jnp.where(kept, slot, 0)
    wf_ref[...] = jnp.where(kept, weight, 0.0)


_gate_route = pl.pallas_call(
    _gate_route_body,
    out_shape=(
        jax.ShapeDtypeStruct((S, 1), jnp.int32),
        jax.ShapeDtypeStruct((S, 1), jnp.int32),
        jax.ShapeDtypeStruct((S, 1), jnp.float32),
    ),
)


_ROWS_D = S // NW


@functools.partial(
    pl.kernel,
    out_type=jax.ShapeDtypeStruct((DISP_PAD, D), jnp.float32),
    mesh=_MESH,
    scratch_types=[
        pltpu.VMEM((_ROWS_D,), jnp.int32),
        pltpu.VMEM((_ROWS_D, D), jnp.float32),
        pltpu.SemaphoreType.DMA,
    ],
)
def _dispatch(tok_hbm, slotd_hbm, disp_hbm, idx_v, rows_v, sem):
    wid = lax.axis_index("s") * NC + lax.axis_index("c")
    base = wid * _ROWS_D
    pltpu.sync_copy(slotd_hbm.at[pl.ds(base, _ROWS_D)], idx_v)
    pltpu.sync_copy(tok_hbm.at[pl.ds(base, _ROWS_D)], rows_v)
    pltpu.async_copy(rows_v, disp_hbm.at[idx_v], sem).wait()


_F_BLK = 768


def _gelu(x):
    c = math.sqrt(2.0 / math.pi)
    return 0.5 * x * (1.0 + jnp.tanh(c * (x + 0.044715 * (x * x * x))))


def _ffn_body(disp_ref, w1_ref, w2_ref, out_ref):
    fb = pl.program_id(1)
    x = disp_ref[...].astype(jnp.bfloat16)
    h = _gelu(jnp.dot(x, w1_ref[0], preferred_element_type=jnp.float32))
    contrib = jnp.dot(h.astype(jnp.bfloat16), w2_ref[0],
                      preferred_element_type=jnp.float32)

    @pl.when(fb == 0)
    def _():
        out_ref[...] = jnp.zeros_like(out_ref)

    out_ref[...] += contrib


_ffn = pl.pallas_call(
    _ffn_body,
    grid=(E, F // _F_BLK),
    in_specs=[
        pl.BlockSpec((C, D), lambda e, fb: (e, 0)),
        pl.BlockSpec((1, D, _F_BLK), lambda e, fb: (e, 0, fb)),
        pl.BlockSpec((1, _F_BLK, D), lambda e, fb: (e, fb, 0)),
    ],
    out_specs=pl.BlockSpec((C, D), lambda e, fb: (e, 0)),
    out_shape=jax.ShapeDtypeStruct((EC, D), jnp.float32),
)


_ROWS_C = S // NW


@functools.partial(
    pl.kernel,
    out_type=jax.ShapeDtypeStruct((S, D), jnp.float32),
    mesh=_MESH,
    scratch_types=[
        pltpu.VMEM((_ROWS_C,), jnp.int32),
        pltpu.VMEM((_ROWS_C,), jnp.float32),
        pltpu.VMEM((_ROWS_C, D), jnp.float32),
        pltpu.SemaphoreType.DMA,
    ],
)
def _combine(eo_hbm, slot_hbm, wf_hbm, out_hbm, idx_v, w_v, rows_v, sem):
    wid = lax.axis_index("s") * NC + lax.axis_index("c")
    base = wid * _ROWS_C
    pltpu.sync_copy(slot_hbm.at[pl.ds(base, _ROWS_C)], idx_v)
    pltpu.sync_copy(wf_hbm.at[pl.ds(base, _ROWS_C)], w_v)
    pltpu.async_copy(eo_hbm.at[idx_v], rows_v, sem).wait()

    def chunk(jj, carry):
        w16 = w_v[pl.ds(jj * 16, 16)]
        for r in range(16):
            wb = _take16(w16, jnp.full((16,), r, jnp.int32))
            i = jj * 16 + r

            def col(k, carry2):
                rows_v[i, pl.ds(k * 16, 16)] = rows_v[i, pl.ds(k * 16, 16)] * wb
                return carry2

            lax.fori_loop(0, D // 16, col, 0)
        return carry

    lax.fori_loop(0, _ROWS_C // 16, chunk, 0)
    pltpu.sync_copy(rows_v, out_hbm.at[pl.ds(base, _ROWS_C)])


def kernel(inputs, gate_w, w1, w2):
    tokens = inputs.reshape(S, D)
    slotd, slot, wf = _gate_route(tokens, gate_w)
    disp = _dispatch(tokens, slotd.reshape(S))
    eo = _ffn(disp, w1.astype(jnp.bfloat16), w2.astype(jnp.bfloat16))
    out = _combine(eo, slot.reshape(S), wf.reshape(S))
    return out.reshape(inputs.shape)


# speedup vs baseline: 1.1464x; 1.0810x over previous
import functools
import math

import jax
import jax.numpy as jnp
from jax import lax
from jax.experimental import pallas as pl
from jax.experimental.pallas import tpu as pltpu
from jax.experimental.pallas import tpu_sc as plsc

S = 2048
D = 768
E = 8
F = 3072
C = 512
EC = E * C
DISP_PAD = EC + 8

NC = 2
NS = 16
NW = NC * NS

_MESH = plsc.VectorSubcoreMesh(
    core_axis_name="c", subcore_axis_name="s", num_cores=NC, num_subcores=NS
)


def _take16(x, idx):
    return lax.gather(
        x, idx[:, None],
        lax.GatherDimensionNumbers(
            offset_dims=(), collapsed_slice_dims=(0,), start_index_map=(0,)),
        slice_sizes=(1,),
        mode=lax.GatherScatterMode.PROMISE_IN_BOUNDS,
    )


def _gate_route_body(tok_ref, gw_ref, slotd_ref, slot_ref, wf_ref):
    logits = lax.dot_general(
        tok_ref[...], gw_ref[...],
        (((1,), (1,)), ((), ())),
        preferred_element_type=jnp.float32,
    )
    lmax = jnp.max(logits, axis=1, keepdims=True)
    wsum = jnp.sum(jnp.exp(logits - lmax), axis=1, keepdims=True)
    weight = 1.0 / wsum
    eidx = lax.broadcasted_iota(jnp.int32, (S, E), 1)
    top1 = jnp.min(jnp.where(logits == lmax, eidx, E), axis=1, keepdims=True)
    oh = (eidx == top1).astype(jnp.float32)
    row = lax.broadcasted_iota(jnp.int32, (S, S), 0)
    col = lax.broadcasted_iota(jnp.int32, (S, S), 1)
    tri = (col <= row).astype(jnp.float32)
    cums = jnp.dot(tri, oh, preferred_element_type=jnp.float32)
    rank = jnp.sum(oh * cums, axis=1, keepdims=True).astype(jnp.int32) - 1
    kept = rank < C
    slot = top1 * C + rank
    slotd_ref[...] = jnp.where(kept, slot, EC)
    slot_ref[...] = jnp.where(kept, slot, 0)
    wf_ref[...] = jnp.where(kept, weight, 0.0)


_gate_route = pl.pallas_call(
    _gate_route_body,
    out_shape=(
        jax.ShapeDtypeStruct((S, 1), jnp.int32),
        jax.ShapeDtypeStruct((S, 1), jnp.int32),
        jax.ShapeDtypeStruct((S, 1), jnp.float32),
    ),
)


_ROWS_D = S // NW


@functools.partial(
    pl.kernel,
    out_type=jax.ShapeDtypeStruct((DISP_PAD, D), jnp.float32),
    mesh=_MESH,
    scratch_types=[
        pltpu.VMEM((_ROWS_D,), jnp.int32),
        pltpu.VMEM((_ROWS_D, D), jnp.float32),
        pltpu.SemaphoreType.DMA,
    ],
)
def _dispatch(tok_hbm, slotd_hbm, disp_hbm, idx_v, rows_v, sem):
    wid = lax.axis_index("s") * NC + lax.axis_index("c")
    base = wid * _ROWS_D
    pltpu.sync_copy(slotd_hbm.at[pl.ds(base, _ROWS_D)], idx_v)
    pltpu.sync_copy(tok_hbm.at[pl.ds(base, _ROWS_D)], rows_v)
    pltpu.async_copy(rows_v, disp_hbm.at[idx_v], sem).wait()


def _gelu(x):
    c = math.sqrt(2.0 / math.pi)
    return x * jax.nn.sigmoid(2.0 * c * (x + 0.044715 * (x * x * x)))


def _ffn_body(disp_ref, w1_ref, w2_ref, out_ref):
    x = disp_ref[...].astype(jnp.bfloat16)
    h = _gelu(jnp.dot(x, w1_ref[0], preferred_element_type=jnp.float32))
    out_ref[...] = jnp.dot(h.astype(jnp.bfloat16), w2_ref[0],
                           preferred_element_type=jnp.float32)


_ffn = pl.pallas_call(
    _ffn_body,
    grid=(E,),
    in_specs=[
        pl.BlockSpec((C, D), lambda e: (e, 0)),
        pl.BlockSpec((1, D, F), lambda e: (e, 0, 0)),
        pl.BlockSpec((1, F, D), lambda e: (e, 0, 0)),
    ],
    out_specs=pl.BlockSpec((C, D), lambda e: (e, 0)),
    out_shape=jax.ShapeDtypeStruct((EC, D), jnp.float32),
)


_ROWS_C = S // NW


@functools.partial(
    pl.kernel,
    out_type=jax.ShapeDtypeStruct((S, D), jnp.float32),
    mesh=_MESH,
    scratch_types=[
        pltpu.VMEM((_ROWS_C,), jnp.int32),
        pltpu.VMEM((_ROWS_C,), jnp.float32),
        pltpu.VMEM((_ROWS_C, D), jnp.float32),
        pltpu.SemaphoreType.DMA,
    ],
)
def _combine(eo_hbm, slot_hbm, wf_hbm, out_hbm, idx_v, w_v, rows_v, sem):
    wid = lax.axis_index("s") * NC + lax.axis_index("c")
    base = wid * _ROWS_C
    pltpu.sync_copy(slot_hbm.at[pl.ds(base, _ROWS_C)], idx_v)
    pltpu.sync_copy(wf_hbm.at[pl.ds(base, _ROWS_C)], w_v)
    pltpu.async_copy(eo_hbm.at[idx_v], rows_v, sem).wait()

    def chunk(jj, carry):
        w16 = w_v[pl.ds(jj * 16, 16)]
        for r in range(16):
            wb = _take16(w16, jnp.full((16,), r, jnp.int32))
            i = jj * 16 + r

            def col(k, carry2):
                rows_v[i, pl.ds(k * 16, 16)] = rows_v[i, pl.ds(k * 16, 16)] * wb
                return carry2

            lax.fori_loop(0, D // 16, col, 0)
        return carry

    lax.fori_loop(0, _ROWS_C // 16, chunk, 0)
    pltpu.sync_copy(rows_v, out_hbm.at[pl.ds(base, _ROWS_C)])


def kernel(inputs, gate_w, w1, w2):
    tokens = inputs.reshape(S, D)
    slotd, slot, wf = _gate_route(tokens, gate_w)
    disp = _dispatch(tokens, slotd.reshape(S))
    eo = _ffn(disp, w1.astype(jnp.bfloat16), w2.astype(jnp.bfloat16))
    out = _combine(eo, slot.reshape(S), wf.reshape(S))
    return out.reshape(inputs.shape)


# speedup vs baseline: 1.9383x; 1.6907x over previous
import functools
import math

import jax
import jax.numpy as jnp
from jax import lax
from jax.experimental import pallas as pl
from jax.experimental.pallas import tpu as pltpu
from jax.experimental.pallas import tpu_sc as plsc

S = 2048
D = 768
E = 8
F = 3072
C = 512
EC = E * C
DISP_PAD = EC + 8

NC = 2
NS = 16
NW = NC * NS

_MESH = plsc.VectorSubcoreMesh(
    core_axis_name="c", subcore_axis_name="s", num_cores=NC, num_subcores=NS
)


def _take16(x, idx):
    return lax.gather(
        x, idx[:, None],
        lax.GatherDimensionNumbers(
            offset_dims=(), collapsed_slice_dims=(0,), start_index_map=(0,)),
        slice_sizes=(1,),
        mode=lax.GatherScatterMode.PROMISE_IN_BOUNDS,
    )


def _gate_route_body(tok_ref, gw_ref, slotd_ref, slot_ref, wf_ref):
    logits = lax.dot_general(
        tok_ref[...], gw_ref[...],
        (((1,), (1,)), ((), ())),
        preferred_element_type=jnp.float32,
    )
    lmax = jnp.max(logits, axis=1, keepdims=True)
    wsum = jnp.sum(jnp.exp(logits - lmax), axis=1, keepdims=True)
    weight = 1.0 / wsum
    eidx = lax.broadcasted_iota(jnp.int32, (S, E), 1)
    top1 = jnp.min(jnp.where(logits == lmax, eidx, E), axis=1, keepdims=True)
    oh = (eidx == top1).astype(jnp.float32)
    row = lax.broadcasted_iota(jnp.int32, (S, S), 0)
    col = lax.broadcasted_iota(jnp.int32, (S, S), 1)
    tri = (col <= row).astype(jnp.float32)
    cums = jnp.dot(tri, oh, preferred_element_type=jnp.float32)
    rank = jnp.sum(oh * cums, axis=1, keepdims=True).astype(jnp.int32) - 1
    kept = rank < C
    slot = top1 * C + rank
    slotd_ref[...] = jnp.where(kept, slot, EC)
    slot_ref[...] = jnp.where(kept, slot, 0)
    wf_ref[...] = jnp.where(kept, weight, 0.0)


_gate_route = pl.pallas_call(
    _gate_route_body,
    out_shape=(
        jax.ShapeDtypeStruct((S, 1), jnp.int32),
        jax.ShapeDtypeStruct((S, 1), jnp.int32),
        jax.ShapeDtypeStruct((S, 1), jnp.float32),
    ),
)


_ROWS_D = S // NW


@functools.partial(
    pl.kernel,
    out_type=jax.ShapeDtypeStruct((DISP_PAD, D), jnp.float32),
    mesh=_MESH,
    scratch_types=[
        pltpu.VMEM((_ROWS_D,), jnp.int32),
        pltpu.VMEM((_ROWS_D, D), jnp.float32),
        pltpu.SemaphoreType.DMA,
    ],
)
def _dispatch(tok_hbm, slotd_hbm, disp_hbm, idx_v, rows_v, sem):
    wid = lax.axis_index("s") * NC + lax.axis_index("c")
    base = wid * _ROWS_D
    pltpu.sync_copy(slotd_hbm.at[pl.ds(base, _ROWS_D)], idx_v)
    pltpu.sync_copy(tok_hbm.at[pl.ds(base, _ROWS_D)], rows_v)
    pltpu.async_copy(rows_v, disp_hbm.at[idx_v], sem).wait()


def _gelu(x):
    c = math.sqrt(2.0 / math.pi)
    return x * jax.nn.sigmoid(2.0 * c * (x + 0.044715 * (x * x * x)))


def _ffn_body(disp_ref, w1_ref, w2_ref, out_ref):
    x = disp_ref[...].astype(jnp.bfloat16)
    w1b = w1_ref[0].astype(jnp.bfloat16)
    h = _gelu(jnp.dot(x, w1b, preferred_element_type=jnp.float32))
    w2b = w2_ref[0].astype(jnp.bfloat16)
    out_ref[...] = jnp.dot(h.astype(jnp.bfloat16), w2b,
                           preferred_element_type=jnp.float32)


_ffn = pl.pallas_call(
    _ffn_body,
    grid=(E,),
    in_specs=[
        pl.BlockSpec((C, D), lambda e: (e, 0)),
        pl.BlockSpec((1, D, F), lambda e: (e, 0, 0)),
        pl.BlockSpec((1, F, D), lambda e: (e, 0, 0)),
    ],
    out_specs=pl.BlockSpec((C, D), lambda e: (e, 0)),
    out_shape=jax.ShapeDtypeStruct((EC, D), jnp.float32),
)


_ROWS_C = S // NW


@functools.partial(
    pl.kernel,
    out_type=jax.ShapeDtypeStruct((S, D), jnp.float32),
    mesh=_MESH,
    scratch_types=[
        pltpu.VMEM((_ROWS_C,), jnp.int32),
        pltpu.VMEM((_ROWS_C,), jnp.float32),
        pltpu.VMEM((_ROWS_C, D), jnp.float32),
        pltpu.SemaphoreType.DMA,
    ],
)
def _combine(eo_hbm, slot_hbm, wf_hbm, out_hbm, idx_v, w_v, rows_v, sem):
    wid = lax.axis_index("s") * NC + lax.axis_index("c")
    base = wid * _ROWS_C
    pltpu.sync_copy(slot_hbm.at[pl.ds(base, _ROWS_C)], idx_v)
    pltpu.sync_copy(wf_hbm.at[pl.ds(base, _ROWS_C)], w_v)
    pltpu.async_copy(eo_hbm.at[idx_v], rows_v, sem).wait()

    def chunk(jj, carry):
        w16 = w_v[pl.ds(jj * 16, 16)]
        for r in range(16):
            wb = _take16(w16, jnp.full((16,), r, jnp.int32))
            i = jj * 16 + r
            for k in range(D // 16):
                rows_v[i, pl.ds(k * 16, 16)] = rows_v[i, pl.ds(k * 16, 16)] * wb
        return carry

    lax.fori_loop(0, _ROWS_C // 16, chunk, 0)
    pltpu.sync_copy(rows_v, out_hbm.at[pl.ds(base, _ROWS_C)])


def kernel(inputs, gate_w, w1, w2):
    tokens = inputs.reshape(S, D)
    slotd, slot, wf = _gate_route(tokens, gate_w)
    disp = _dispatch(tokens, slotd.reshape(S))
    eo = _ffn(disp, w1, w2)
    out = _combine(eo, slot.reshape(S), wf.reshape(S))
    return out.reshape(inputs.shape)


# speedup vs baseline: 1.9606x; 1.0115x over previous
import functools
import math

import jax
import jax.numpy as jnp
from jax import lax
from jax.experimental import pallas as pl
from jax.experimental.pallas import tpu as pltpu
from jax.experimental.pallas import tpu_sc as plsc

S = 2048
D = 768
E = 8
F = 3072
C = 512
EC = E * C
DISP_PAD = EC + 8

NC = 2
NS = 16
NW = NC * NS

_MESH = plsc.VectorSubcoreMesh(
    core_axis_name="c", subcore_axis_name="s", num_cores=NC, num_subcores=NS
)


def _take16(x, idx):
    return lax.gather(
        x, idx[:, None],
        lax.GatherDimensionNumbers(
            offset_dims=(), collapsed_slice_dims=(0,), start_index_map=(0,)),
        slice_sizes=(1,),
        mode=lax.GatherScatterMode.PROMISE_IN_BOUNDS,
    )


def _gate_route_body(tok_ref, gw_ref, slotd_ref, slot_ref, wf_ref):
    logits = lax.dot_general(
        tok_ref[...], gw_ref[...],
        (((1,), (1,)), ((), ())),
        preferred_element_type=jnp.float32,
    )
    lmax = jnp.max(logits, axis=1, keepdims=True)
    wsum = jnp.sum(jnp.exp(logits - lmax), axis=1, keepdims=True)
    weight = 1.0 / wsum
    eidx = lax.broadcasted_iota(jnp.int32, (S, E), 1)
    top1 = jnp.min(jnp.where(logits == lmax, eidx, E), axis=1, keepdims=True)
    oh = (eidx == top1).astype(jnp.float32)
    row = lax.broadcasted_iota(jnp.int32, (S, S), 0)
    col = lax.broadcasted_iota(jnp.int32, (S, S), 1)
    tri = (col <= row).astype(jnp.float32)
    cums = jnp.dot(tri, oh, preferred_element_type=jnp.float32)
    rank = jnp.sum(oh * cums, axis=1, keepdims=True).astype(jnp.int32) - 1
    kept = rank < C
    slot = top1 * C + rank
    slotd_ref[...] = jnp.where(kept, slot, EC)
    slot_ref[...] = jnp.where(kept, slot, 0)
    wf_ref[...] = jnp.where(kept, weight, 0.0)


_gate_route = pl.pallas_call(
    _gate_route_body,
    out_shape=(
        jax.ShapeDtypeStruct((S, 1), jnp.int32),
        jax.ShapeDtypeStruct((S, 1), jnp.int32),
        jax.ShapeDtypeStruct((S, 1), jnp.float32),
    ),
)


_ROWS_D = S // NW


@functools.partial(
    pl.kernel,
    out_type=jax.ShapeDtypeStruct((DISP_PAD, D), jnp.float32),
    mesh=_MESH,
    scratch_types=[
        pltpu.VMEM((_ROWS_D,), jnp.int32),
        pltpu.VMEM((_ROWS_D, D), jnp.float32),
        pltpu.SemaphoreType.DMA,
    ],
)
def _dispatch(tok_hbm, slotd_hbm, disp_hbm, idx_v, rows_v, sem):
    wid = lax.axis_index("s") * NC + lax.axis_index("c")
    base = wid * _ROWS_D
    pltpu.sync_copy(slotd_hbm.at[pl.ds(base, _ROWS_D)], idx_v)
    pltpu.sync_copy(tok_hbm.at[pl.ds(base, _ROWS_D)], rows_v)
    pltpu.async_copy(rows_v, disp_hbm.at[idx_v], sem).wait()


def _gelu(x):
    c = math.sqrt(2.0 / math.pi)
    return x * jax.nn.sigmoid(2.0 * c * (x + 0.044715 * (x * x * x)))


def _ffn_body(disp_ref, w1_ref, w2_ref, out_ref):
    x = disp_ref[...].astype(jnp.bfloat16)
    w1b = w1_ref[0].astype(jnp.bfloat16)
    h = jnp.dot(x, w1b, preferred_element_type=jnp.float32).astype(jnp.bfloat16)
    g = _gelu(h)
    w2b = w2_ref[0].astype(jnp.bfloat16)
    out_ref[...] = jnp.dot(g, w2b, preferred_element_type=jnp.float32)


_ffn = pl.pallas_call(
    _ffn_body,
    grid=(E,),
    in_specs=[
        pl.BlockSpec((C, D), lambda e: (e, 0)),
        pl.BlockSpec((1, D, F), lambda e: (e, 0, 0)),
        pl.BlockSpec((1, F, D), lambda e: (e, 0, 0)),
    ],
    out_specs=pl.BlockSpec((C, D), lambda e: (e, 0)),
    out_shape=jax.ShapeDtypeStruct((EC, D), jnp.float32),
)


_ROWS_C = S // NW


@functools.partial(
    pl.kernel,
    out_type=jax.ShapeDtypeStruct((S, D), jnp.float32),
    mesh=_MESH,
    scratch_types=[
        pltpu.VMEM((_ROWS_C,), jnp.int32),
        pltpu.VMEM((_ROWS_C,), jnp.float32),
        pltpu.VMEM((_ROWS_C, D), jnp.float32),
        pltpu.SemaphoreType.DMA,
    ],
)
def _combine(eo_hbm, slot_hbm, wf_hbm, out_hbm, idx_v, w_v, rows_v, sem):
    wid = lax.axis_index("s") * NC + lax.axis_index("c")
    base = wid * _ROWS_C
    pltpu.sync_copy(slot_hbm.at[pl.ds(base, _ROWS_C)], idx_v)
    pltpu.sync_copy(wf_hbm.at[pl.ds(base, _ROWS_C)], w_v)
    pltpu.async_copy(eo_hbm.at[idx_v], rows_v, sem).wait()

    def chunk(jj, carry):
        w16 = w_v[pl.ds(jj * 16, 16)]
        for r in range(16):
            wb = _take16(w16, jnp.full((16,), r, jnp.int32))
            i = jj * 16 + r
            for k in range(D // 16):
                rows_v[i, pl.ds(k * 16, 16)] = rows_v[i, pl.ds(k * 16, 16)] * wb
        return carry

    lax.fori_loop(0, _ROWS_C // 16, chunk, 0)
    pltpu.sync_copy(rows_v, out_hbm.at[pl.ds(base, _ROWS_C)])


def kernel(inputs, gate_w, w1, w2):
    tokens = inputs.reshape(S, D)
    slotd, slot, wf = _gate_route(tokens, gate_w)
    disp = _dispatch(tokens, slotd.reshape(S))
    eo = _ffn(disp, w1, w2)
    out = _combine(eo, slot.reshape(S), wf.reshape(S))
    return out.reshape(inputs.shape)


# speedup vs baseline: 2.0885x; 1.0652x over previous
import functools
import math

import jax
import jax.numpy as jnp
from jax import lax
from jax.experimental import pallas as pl
from jax.experimental.pallas import tpu as pltpu
from jax.experimental.pallas import tpu_sc as plsc

S = 2048
D = 768
E = 8
F = 3072
C = 512
EC = E * C
DISP_PAD = EC + 8

NC = 2
NS = 16
NW = NC * NS

_MESH = plsc.VectorSubcoreMesh(
    core_axis_name="c", subcore_axis_name="s", num_cores=NC, num_subcores=NS
)


def _take16(x, idx):
    return lax.gather(
        x, idx[:, None],
        lax.GatherDimensionNumbers(
            offset_dims=(), collapsed_slice_dims=(0,), start_index_map=(0,)),
        slice_sizes=(1,),
        mode=lax.GatherScatterMode.PROMISE_IN_BOUNDS,
    )


def _gate_route_body(tok_ref, gw_ref, slotd_ref, slot_ref, wf_ref):
    logits = lax.dot_general(
        gw_ref[...], tok_ref[...],
        (((1,), (1,)), ((), ())),
        preferred_element_type=jnp.float32,
    )
    lmax = jnp.max(logits, axis=0, keepdims=True)
    wsum = jnp.sum(jnp.exp(logits - lmax), axis=0, keepdims=True)
    weight = 1.0 / wsum
    eidx = lax.broadcasted_iota(jnp.int32, (E, S), 0)
    top1 = jnp.min(jnp.where(logits == lmax, eidx, E), axis=0, keepdims=True)
    oh = (eidx == top1).astype(jnp.float32)
    row = lax.broadcasted_iota(jnp.int32, (S, S), 0)
    col = lax.broadcasted_iota(jnp.int32, (S, S), 1)
    tri = (row <= col).astype(jnp.float32)
    cums = jnp.dot(oh, tri, preferred_element_type=jnp.float32)
    rank = jnp.sum(oh * cums, axis=0, keepdims=True).astype(jnp.int32) - 1
    kept = rank < C
    slot = top1 * C + rank
    slotd_ref[...] = jnp.where(kept, slot, EC).reshape(S)
    slot_ref[...] = jnp.where(kept, slot, 0).reshape(S)
    wf_ref[...] = jnp.where(kept, weight, 0.0).reshape(S)


_gate_route = pl.pallas_call(
    _gate_route_body,
    out_shape=(
        jax.ShapeDtypeStruct((S,), jnp.int32),
        jax.ShapeDtypeStruct((S,), jnp.int32),
        jax.ShapeDtypeStruct((S,), jnp.float32),
    ),
)


_ROWS_D = S // NW


@functools.partial(
    pl.kernel,
    out_type=jax.ShapeDtypeStruct((DISP_PAD, D), jnp.float32),
    mesh=_MESH,
    scratch_types=[
        pltpu.VMEM((_ROWS_D,), jnp.int32),
        pltpu.VMEM((_ROWS_D, D), jnp.float32),
        pltpu.SemaphoreType.DMA,
    ],
)
def _dispatch(tok_hbm, slotd_hbm, disp_hbm, idx_v, rows_v, sem):
    wid = lax.axis_index("s") * NC + lax.axis_index("c")
    base = wid * _ROWS_D
    pltpu.sync_copy(slotd_hbm.at[pl.ds(base, _ROWS_D)], idx_v)
    pltpu.sync_copy(tok_hbm.at[pl.ds(base, _ROWS_D)], rows_v)
    pltpu.async_copy(rows_v, disp_hbm.at[idx_v], sem).wait()


def _gelu(x):
    c = math.sqrt(2.0 / math.pi)
    return x * jax.nn.sigmoid(2.0 * c * (x + 0.044715 * (x * x * x)))


def _ffn_body(disp_ref, w1_ref, w2_ref, out_ref):
    x = disp_ref[...].astype(jnp.bfloat16)
    w1b = w1_ref[0].astype(jnp.bfloat16)
    h = jnp.dot(x, w1b, preferred_element_type=jnp.float32).astype(jnp.bfloat16)
    g = _gelu(h)
    w2b = w2_ref[0].astype(jnp.bfloat16)
    out_ref[...] = jnp.dot(g, w2b, preferred_element_type=jnp.float32)


_ffn = pl.pallas_call(
    _ffn_body,
    grid=(E,),
    in_specs=[
        pl.BlockSpec((C, D), lambda e: (e, 0)),
        pl.BlockSpec((1, D, F), lambda e: (e, 0, 0)),
        pl.BlockSpec((1, F, D), lambda e: (e, 0, 0)),
    ],
    out_specs=pl.BlockSpec((C, D), lambda e: (e, 0)),
    out_shape=jax.ShapeDtypeStruct((EC, D), jnp.float32),
)


_ROWS_C = S // NW


@functools.partial(
    pl.kernel,
    out_type=jax.ShapeDtypeStruct((S, D), jnp.float32),
    mesh=_MESH,
    scratch_types=[
        pltpu.VMEM((_ROWS_C,), jnp.int32),
        pltpu.VMEM((_ROWS_C,), jnp.float32),
        pltpu.VMEM((_ROWS_C, D), jnp.float32),
        pltpu.SemaphoreType.DMA,
    ],
)
def _combine(eo_hbm, slot_hbm, wf_hbm, out_hbm, idx_v, w_v, rows_v, sem):
    wid = lax.axis_index("s") * NC + lax.axis_index("c")
    base = wid * _ROWS_C
    pltpu.sync_copy(slot_hbm.at[pl.ds(base, _ROWS_C)], idx_v)
    pltpu.sync_copy(wf_hbm.at[pl.ds(base, _ROWS_C)], w_v)
    pltpu.async_copy(eo_hbm.at[idx_v], rows_v, sem).wait()

    def chunk(jj, carry):
        w16 = w_v[pl.ds(jj * 16, 16)]
        for r in range(16):
            wb = _take16(w16, jnp.full((16,), r, jnp.int32))
            i = jj * 16 + r
            for k in range(D // 16):
                rows_v[i, pl.ds(k * 16, 16)] = rows_v[i, pl.ds(k * 16, 16)] * wb
        return carry

    lax.fori_loop(0, _ROWS_C // 16, chunk, 0)
    pltpu.sync_copy(rows_v, out_hbm.at[pl.ds(base, _ROWS_C)])


def kernel(inputs, gate_w, w1, w2):
    tokens = inputs.reshape(S, D)
    slotd, slot, wf = _gate_route(tokens, gate_w)
    disp = _dispatch(tokens, slotd)
    eo = _ffn(disp, w1, w2)
    out = _combine(eo, slot, wf)
    return out.reshape(inputs.shape)
